# Initial kernel scaffold; baseline (speedup 1.0000x reference)
import jax
import jax.numpy as jnp
from jax.experimental import pallas as pl


def kernel(user, positive, negative, edge_index, edge_weight, user_emb_w, item_emb_w):
    raise NotImplementedError("write your pallas kernel here")


# speedup vs baseline: 6.8590x; 6.8590x over previous
import functools

import jax
import jax.numpy as jnp
from jax import lax
from jax.experimental import pallas as pl
from jax.experimental.pallas import tpu as pltpu
from jax.experimental.pallas import tpu_sc as plsc

NU = 50000
NI = 50000
NN = NU + NI
D = 32
E = 1600000
EH = E // 2
CH = 128
NCH = EH // CH
NS = 16
WB = 128
NWB = NU // WB
WB_TAIL = NU - NWB * WB
B = 4096
BR = 256
NBLK = B // BR
INV_T = 5.0
MARGIN = 0.1
L_REG = 1e-4

_mesh = plsc.VectorSubcoreMesh(core_axis_name="c", subcore_axis_name="s")


@functools.partial(
    pl.kernel,
    out_type=jax.ShapeDtypeStruct((NN, D), jnp.float32),
    mesh=_mesh,
    compiler_params=pltpu.CompilerParams(use_tc_tiling_on_sc=False),
    scratch_types=[
        pltpu.VMEM((1, CH), jnp.int32),
        pltpu.VMEM((1, CH), jnp.int32),
        pltpu.VMEM((1, CH), jnp.float32),
        pltpu.VMEM((CH, D), jnp.float32),
        pltpu.VMEM((WB, D), jnp.float32),
        pltpu.VMEM_SHARED((NU, D), jnp.float32),
        pltpu.SemaphoreType.DMA,
    ],
)
def _spmm(rowi, coli, w, prev, out, rowb, colb, wb, rows, wrb, acc, sem):
    cid = lax.axis_index("c")
    sid = lax.axis_index("s")

    zero = jnp.zeros((16,), jnp.float32)

    def _z1(i, _):
        wrb[i // 2, pl.ds((i % 2) * 16, 16)] = zero
        return 0

    lax.fori_loop(0, WB * 2, _z1, 0)

    nwb = 24 + jnp.where(sid < NWB - 24 * NS, 1, 0)

    def _z2(j, _):
        pltpu.sync_copy(wrb, acc.at[pl.ds((sid + j * NS) * WB, WB), :])
        return 0

    lax.fori_loop(0, nwb, _z2, 0)

    @pl.when(sid == NS - 1)
    def _():
        pltpu.sync_copy(wrb.at[pl.ds(0, WB_TAIL), :],
                        acc.at[pl.ds(NWB * WB, WB_TAIL), :])

    plsc.subcore_barrier()

    nch = 390 + jnp.where(sid < NCH - 390 * NS, 1, 0)
    off = cid * NU

    def _chunk(k, _):
        c = sid + k * NS
        b0 = cid * EH + c * CH
        pltpu.sync_copy(rowi.at[pl.ds(b0, CH)], rowb.at[0])
        pltpu.sync_copy(coli.at[pl.ds(b0, CH)], colb.at[0])
        pltpu.sync_copy(w.at[pl.ds(b0, CH)], wb.at[0])

        def _sub(i, _):
            rowb[0, pl.ds(i * 16, 16)] = rowb[0, pl.ds(i * 16, 16)] - off
            return 0

        lax.fori_loop(0, CH // 16, _sub, 0)
        pltpu.async_copy(prev.at[colb.at[0]], rows, sem).wait()

        def _scale(gi, _):
            wv = wb[0, pl.ds(gi * 16, 16)]
            for j in range(16):
                e = gi * 16 + j
                w = wv[j]
                rows[e, pl.ds(0, 16)] = rows[e, pl.ds(0, 16)] * w
                rows[e, pl.ds(16, 16)] = rows[e, pl.ds(16, 16)] * w
            return 0

        lax.fori_loop(0, CH // 16, _scale, 0)
        pltpu.sync_copy(rows, acc.at[rowb.at[0]], add=True)
        return 0

    lax.fori_loop(0, nch, _chunk, 0)
    plsc.subcore_barrier()

    def _wbk(j, _):
        r0 = (sid + j * NS) * WB
        pltpu.sync_copy(acc.at[pl.ds(r0, WB), :], wrb)
        pltpu.sync_copy(wrb, out.at[pl.ds(cid * NU + r0, WB), :])
        return 0

    lax.fori_loop(0, nwb, _wbk, 0)

    @pl.when(sid == NS - 1)
    def _():
        pltpu.sync_copy(acc.at[pl.ds(NWB * WB, WB_TAIL), :],
                        wrb.at[pl.ds(0, WB_TAIL), :])
        pltpu.sync_copy(wrb.at[pl.ds(0, WB_TAIL), :],
                        out.at[pl.ds(cid * NU + NWB * WB, WB_TAIL), :])


_BPT = B // (2 * NS)


@functools.partial(
    pl.kernel,
    out_type=[jax.ShapeDtypeStruct((B, D), jnp.float32) for _ in range(5)],
    mesh=_mesh,
    compiler_params=pltpu.CompilerParams(use_tc_tiling_on_sc=False),
    scratch_types=[
        pltpu.VMEM((_BPT,), jnp.int32),
        pltpu.VMEM((_BPT, D), jnp.float32),
        pltpu.VMEM((_BPT, D), jnp.float32),
        pltpu.SemaphoreType.DMA,
    ],
)
def _gather_mean(e0, e1, e2, e3, user, positive, negative,
                 user_e, pos_e, ego_u, ego_p, ego_n, idxb, rb, accb, sem):
    cid = lax.axis_index("c")
    sid = lax.axis_index("s")
    wid = sid * 2 + cid
    base = wid * _BPT
    tables = [e0, e1, e2, e3]

    def _acc_from_rb(first):
        def _body(i, _):
            r = i // 2
            s = (i % 2) * 16
            v = rb[r, pl.ds(s, 16)]
            if first:
                accb[r, pl.ds(s, 16)] = v
            else:
                accb[r, pl.ds(s, 16)] = accb[r, pl.ds(s, 16)] + v
            return 0
        lax.fori_loop(0, _BPT * 2, _body, 0)

    def _scale_acc():
        def _body(i, _):
            r = i // 2
            s = (i % 2) * 16
            accb[r, pl.ds(s, 16)] = accb[r, pl.ds(s, 16)] * 0.25
            return 0
        lax.fori_loop(0, _BPT * 2, _body, 0)

    def _shift_idx(delta):
        def _body(i, _):
            idxb[pl.ds(i * 16, 16)] = idxb[pl.ds(i * 16, 16)] + delta
            return 0
        lax.fori_loop(0, _BPT // 16, _body, 0)

    pltpu.sync_copy(user.at[pl.ds(base, _BPT)], idxb)
    for li, t in enumerate(tables):
        pltpu.async_copy(t.at[idxb], rb, sem).wait()
        if li == 0:
            pltpu.sync_copy(rb, ego_u.at[pl.ds(base, _BPT), :])
        _acc_from_rb(first=(li == 0))
    _scale_acc()
    pltpu.sync_copy(accb, user_e.at[pl.ds(base, _BPT), :])

    pltpu.sync_copy(positive.at[pl.ds(base, _BPT)], idxb)
    _shift_idx(NU)
    for li, t in enumerate(tables):
        pltpu.async_copy(t.at[idxb], rb, sem).wait()
        if li == 0:
            pltpu.sync_copy(rb, ego_p.at[pl.ds(base, _BPT), :])
        _acc_from_rb(first=(li == 0))
    _scale_acc()
    pltpu.sync_copy(accb, pos_e.at[pl.ds(base, _BPT), :])

    pltpu.sync_copy(negative.at[pl.ds(base, _BPT)], idxb)
    _shift_idx(NU)
    pltpu.async_copy(e0.at[idxb], rb, sem).wait()
    pltpu.sync_copy(rb, ego_n.at[pl.ds(base, _BPT), :])


def _loss_body(ue_b, pe_b, ue_f, pe_f, eu, ep, en, reg_ref, na_ref):
    i = pl.program_id(0)

    def _nrm(x):
        n = jnp.maximum(jnp.sqrt(jnp.sum(x * x, axis=1, keepdims=True)), 1e-12)
        return x / n

    e1nb = _nrm(ue_b[...])
    e2nb = _nrm(pe_b[...])
    bfull = _nrm(ue_f[...]) + _nrm(pe_f[...])
    t = lax.dot_general(e1nb, bfull, (((1,), (1,)), ((), ())),
                        preferred_element_type=jnp.float32,
                        precision=lax.Precision.HIGHEST)
    f = jnp.exp(t * INV_T) + jnp.exp(jnp.maximum(t - MARGIN, 0.0) * INV_T)
    tot = jnp.sum(f, axis=1)
    sim = jnp.sum(e1nb * e2nb, axis=1)
    pos = jnp.exp(sim * INV_T) + jnp.exp(jnp.maximum(sim - MARGIN, 0.0) * INV_T)
    part = jnp.sum(-jnp.log(pos / tot + 1e-5))

    @pl.when(i == 0)
    def _():
        na_ref[...] = jnp.zeros((1, 1), jnp.float32)

    na_ref[...] = na_ref[...] + part.reshape(1, 1)

    @pl.when(i == NBLK - 1)
    def _():
        na_ref[...] = na_ref[...] * (1.0 / B)
        reg = (L_REG * 0.5 / B) * (
            jnp.sum(eu[...] ** 2) + jnp.sum(ep[...] ** 2) + jnp.sum(en[...] ** 2))
        reg_ref[...] = reg.reshape(1, 1)


def _loss_tc(ue, pe, eu, ep, en):
    full = pl.BlockSpec((B, D), lambda i: (0, 0))
    blk = pl.BlockSpec((BR, D), lambda i: (i, 0))
    scal = pl.BlockSpec((1, 1), lambda i: (0, 0))
    return pl.pallas_call(
        _loss_body,
        grid=(NBLK,),
        in_specs=[blk, blk, full, full, full, full, full],
        out_specs=[scal, scal],
        out_shape=[jax.ShapeDtypeStruct((1, 1), jnp.float32),
                   jax.ShapeDtypeStruct((1, 1), jnp.float32)],
    )(ue, pe, ue, pe, eu, ep, en)


def kernel(user, positive, negative, edge_index, edge_weight, user_emb_w, item_emb_w):
    e0 = jnp.concatenate([user_emb_w, item_emb_w], axis=0)
    rowi = edge_index[0]
    coli = edge_index[1]
    e1 = _spmm(rowi, coli, edge_weight, e0)
    e2 = _spmm(rowi, coli, edge_weight, e1)
    e3 = _spmm(rowi, coli, edge_weight, e2)
    ue, pe, eu, ep, en = _gather_mean(e0, e1, e2, e3, user, positive, negative)
    reg, na = _loss_tc(ue, pe, eu, ep, en)
    return (reg[0, 0], na[0, 0])


# speedup vs baseline: 16.0728x; 2.3433x over previous
import functools

import jax
import jax.numpy as jnp
from jax import lax
from jax.experimental import pallas as pl
from jax.experimental.pallas import tpu as pltpu
from jax.experimental.pallas import tpu_sc as plsc

NU = 50000
NI = 50000
NN = NU + NI
D = 32
E = 1600000
EH = E // 2
CH = 128
NCH = EH // CH
G = 8
NFG = 48
NS = 16
WB = 128
NWB = NU // WB
WB_TAIL = NU - NWB * WB
B = 4096
BR = 256
NBLK = B // BR
INV_T = 5.0
MARGIN = 0.1
L_REG = 1e-4

_mesh = plsc.VectorSubcoreMesh(core_axis_name="c", subcore_axis_name="s")


@functools.partial(
    pl.kernel,
    out_type=jax.ShapeDtypeStruct((NN, D), jnp.float32),
    mesh=_mesh,
    compiler_params=pltpu.CompilerParams(use_tc_tiling_on_sc=False),
    scratch_types=[
        pltpu.VMEM((G, CH), jnp.int32),
        pltpu.VMEM((G, CH), jnp.int32),
        pltpu.VMEM((G, CH), jnp.float32),
        pltpu.VMEM((2, CH, D), jnp.float32),
        pltpu.VMEM((WB, D), jnp.float32),
        pltpu.VMEM_SHARED((NU, D), jnp.float32),
        pltpu.SemaphoreType.DMA,
        pltpu.SemaphoreType.DMA,
    ],
)
def _spmm(row2d, col2d, w2d, prev, out, rowg, colg, wg, rows, wrb, acc,
          sem0, sem1):
    cid = lax.axis_index("c")
    sid = lax.axis_index("s")

    zero = jnp.zeros((16,), jnp.float32)

    def _z1(i, _):
        wrb[i // 2, pl.ds((i % 2) * 16, 16)] = zero
        return 0

    lax.fori_loop(0, WB * 2, _z1, 0)

    nwb = 24 + jnp.where(sid < NWB - 24 * NS, 1, 0)

    def _z2(j, _):
        pltpu.sync_copy(wrb, acc.at[pl.ds((sid + j * NS) * WB, WB), :])
        return 0

    lax.fori_loop(0, nwb, _z2, 0)

    @pl.when(sid == NS - 1)
    def _():
        pltpu.sync_copy(wrb.at[pl.ds(0, WB_TAIL), :],
                        acc.at[pl.ds(NWB * WB, WB_TAIL), :])

    plsc.subcore_barrier()

    n = 390 + jnp.where(sid < NCH - 390 * NS, 1, 0)
    base_c = sid * 390 + jnp.minimum(sid, NCH - 390 * NS)
    off = cid * NU

    def _scale(j, p):
        def _body(s, _):
            wv = wg[j, pl.ds(s * 16, 16)]
            for q in range(16):
                wq = wv[q]
                e = s * 16 + q
                rows[p, e, pl.ds(0, 16)] = rows[p, e, pl.ds(0, 16)] * wq
                rows[p, e, pl.ds(16, 16)] = rows[p, e, pl.ds(16, 16)] * wq
            return 0
        lax.fori_loop(0, CH // 16, _body, 0)

    def _localize(ng):
        def _body(i, _):
            r = i // (CH // 16)
            s = (i % (CH // 16)) * 16
            rowg[r, pl.ds(s, 16)] = rowg[r, pl.ds(s, 16)] - off
            return 0
        lax.fori_loop(0, ng * (CH // 16), _body, 0)

    sems = (sem0, sem1)

    def _group(gi, _):
        cg = cid * NCH + base_c + gi * G
        pltpu.sync_copy(row2d.at[pl.ds(cg, G), :], rowg)
        pltpu.sync_copy(col2d.at[pl.ds(cg, G), :], colg)
        pltpu.sync_copy(w2d.at[pl.ds(cg, G), :], wg)
        _localize(G)
        h = pltpu.async_copy(prev.at[colg.at[0]], rows.at[0], sems[0])
        for j in range(G):
            if j + 1 < G:
                hn = pltpu.async_copy(prev.at[colg.at[j + 1]],
                                      rows.at[(j + 1) % 2], sems[(j + 1) % 2])
            h.wait()
            _scale(j, j % 2)
            pltpu.sync_copy(rows.at[j % 2], acc.at[rowg.at[j]], add=True)
            if j + 1 < G:
                h = hn
        return 0

    lax.fori_loop(0, NFG, _group, 0)

    def _tail(k, _):
        cg = cid * NCH + base_c + NFG * G + k
        pltpu.sync_copy(row2d.at[cg], rowg.at[0])
        pltpu.sync_copy(col2d.at[cg], colg.at[0])
        pltpu.sync_copy(w2d.at[cg], wg.at[0])
        _localize(1)
        pltpu.async_copy(prev.at[colg.at[0]], rows.at[0], sem0).wait()
        _scale(0, 0)
        pltpu.sync_copy(rows.at[0], acc.at[rowg.at[0]], add=True)
        return 0

    lax.fori_loop(0, n - NFG * G, _tail, 0)
    plsc.subcore_barrier()

    def _wbk(j, _):
        r0 = (sid + j * NS) * WB
        pltpu.sync_copy(acc.at[pl.ds(r0, WB), :], wrb)
        pltpu.sync_copy(wrb, out.at[pl.ds(cid * NU + r0, WB), :])
        return 0

    lax.fori_loop(0, nwb, _wbk, 0)

    @pl.when(sid == NS - 1)
    def _():
        pltpu.sync_copy(acc.at[pl.ds(NWB * WB, WB_TAIL), :],
                        wrb.at[pl.ds(0, WB_TAIL), :])
        pltpu.sync_copy(wrb.at[pl.ds(0, WB_TAIL), :],
                        out.at[pl.ds(cid * NU + NWB * WB, WB_TAIL), :])


_BPT = B // (2 * NS)


@functools.partial(
    pl.kernel,
    out_type=[jax.ShapeDtypeStruct((B, D), jnp.float32) for _ in range(5)],
    mesh=_mesh,
    compiler_params=pltpu.CompilerParams(use_tc_tiling_on_sc=False),
    scratch_types=[
        pltpu.VMEM((_BPT,), jnp.int32),
        pltpu.VMEM((_BPT, D), jnp.float32),
        pltpu.VMEM((_BPT, D), jnp.float32),
        pltpu.SemaphoreType.DMA,
    ],
)
def _gather_mean(e0, e1, e2, e3, user, positive, negative,
                 user_e, pos_e, ego_u, ego_p, ego_n, idxb, rb, accb, sem):
    cid = lax.axis_index("c")
    sid = lax.axis_index("s")
    wid = sid * 2 + cid
    base = wid * _BPT
    tables = [e0, e1, e2, e3]

    def _acc_from_rb(first):
        def _body(i, _):
            r = i // 2
            s = (i % 2) * 16
            v = rb[r, pl.ds(s, 16)]
            if first:
                accb[r, pl.ds(s, 16)] = v
            else:
                accb[r, pl.ds(s, 16)] = accb[r, pl.ds(s, 16)] + v
            return 0
        lax.fori_loop(0, _BPT * 2, _body, 0)

    def _scale_acc():
        def _body(i, _):
            r = i // 2
            s = (i % 2) * 16
            accb[r, pl.ds(s, 16)] = accb[r, pl.ds(s, 16)] * 0.25
            return 0
        lax.fori_loop(0, _BPT * 2, _body, 0)

    def _shift_idx(delta):
        def _body(i, _):
            idxb[pl.ds(i * 16, 16)] = idxb[pl.ds(i * 16, 16)] + delta
            return 0
        lax.fori_loop(0, _BPT // 16, _body, 0)

    pltpu.sync_copy(user.at[pl.ds(base, _BPT)], idxb)
    for li, t in enumerate(tables):
        pltpu.async_copy(t.at[idxb], rb, sem).wait()
        if li == 0:
            pltpu.sync_copy(rb, ego_u.at[pl.ds(base, _BPT), :])
        _acc_from_rb(first=(li == 0))
    _scale_acc()
    pltpu.sync_copy(accb, user_e.at[pl.ds(base, _BPT), :])

    pltpu.sync_copy(positive.at[pl.ds(base, _BPT)], idxb)
    _shift_idx(NU)
    for li, t in enumerate(tables):
        pltpu.async_copy(t.at[idxb], rb, sem).wait()
        if li == 0:
            pltpu.sync_copy(rb, ego_p.at[pl.ds(base, _BPT), :])
        _acc_from_rb(first=(li == 0))
    _scale_acc()
    pltpu.sync_copy(accb, pos_e.at[pl.ds(base, _BPT), :])

    pltpu.sync_copy(negative.at[pl.ds(base, _BPT)], idxb)
    _shift_idx(NU)
    pltpu.async_copy(e0.at[idxb], rb, sem).wait()
    pltpu.sync_copy(rb, ego_n.at[pl.ds(base, _BPT), :])


def _loss_body(ue_b, pe_b, ue_f, pe_f, eu, ep, en, reg_ref, na_ref):
    i = pl.program_id(0)

    def _nrm(x):
        n = jnp.maximum(jnp.sqrt(jnp.sum(x * x, axis=1, keepdims=True)), 1e-12)
        return x / n

    e1nb = _nrm(ue_b[...])
    e2nb = _nrm(pe_b[...])
    bfull = _nrm(ue_f[...]) + _nrm(pe_f[...])
    t = lax.dot_general(e1nb, bfull, (((1,), (1,)), ((), ())),
                        preferred_element_type=jnp.float32,
                        precision=lax.Precision.HIGHEST)
    f = jnp.exp(t * INV_T) + jnp.exp(jnp.maximum(t - MARGIN, 0.0) * INV_T)
    tot = jnp.sum(f, axis=1)
    sim = jnp.sum(e1nb * e2nb, axis=1)
    pos = jnp.exp(sim * INV_T) + jnp.exp(jnp.maximum(sim - MARGIN, 0.0) * INV_T)
    part = jnp.sum(-jnp.log(pos / tot + 1e-5))

    @pl.when(i == 0)
    def _():
        na_ref[...] = jnp.zeros((1, 1), jnp.float32)

    na_ref[...] = na_ref[...] + part.reshape(1, 1)

    @pl.when(i == NBLK - 1)
    def _():
        na_ref[...] = na_ref[...] * (1.0 / B)
        reg = (L_REG * 0.5 / B) * (
            jnp.sum(eu[...] ** 2) + jnp.sum(ep[...] ** 2) + jnp.sum(en[...] ** 2))
        reg_ref[...] = reg.reshape(1, 1)


def _loss_tc(ue, pe, eu, ep, en):
    full = pl.BlockSpec((B, D), lambda i: (0, 0))
    blk = pl.BlockSpec((BR, D), lambda i: (i, 0))
    scal = pl.BlockSpec((1, 1), lambda i: (0, 0))
    return pl.pallas_call(
        _loss_body,
        grid=(NBLK,),
        in_specs=[blk, blk, full, full, full, full, full],
        out_specs=[scal, scal],
        out_shape=[jax.ShapeDtypeStruct((1, 1), jnp.float32),
                   jax.ShapeDtypeStruct((1, 1), jnp.float32)],
    )(ue, pe, ue, pe, eu, ep, en)


def kernel(user, positive, negative, edge_index, edge_weight, user_emb_w, item_emb_w):
    e0 = jnp.concatenate([user_emb_w, item_emb_w], axis=0)
    row2d = edge_index[0].reshape(2 * NCH, CH)
    col2d = edge_index[1].reshape(2 * NCH, CH)
    w2d = edge_weight.reshape(2 * NCH, CH)
    e1 = _spmm(row2d, col2d, w2d, e0)
    e2 = _spmm(row2d, col2d, w2d, e1)
    e3 = _spmm(row2d, col2d, w2d, e2)
    ue, pe, eu, ep, en = _gather_mean(e0, e1, e2, e3, user, positive, negative)
    reg, na = _loss_tc(ue, pe, eu, ep, en)
    return (reg[0, 0], na[0, 0])


# speedup vs baseline: 17.4349x; 1.0847x over previous
import functools

import jax
import jax.numpy as jnp
from jax import lax
from jax.experimental import pallas as pl
from jax.experimental.pallas import tpu as pltpu
from jax.experimental.pallas import tpu_sc as plsc

NU = 50000
NI = 50000
NN = NU + NI
D = 32
E = 1600000
EH = E // 2
CH = 128
NCH = EH // CH
G = 8
NFG = 48
NS = 16
WB = 128
NWB = NU // WB
WB_TAIL = NU - NWB * WB
B = 4096
BR = 256
NBLK = B // BR
INV_T = 5.0
MARGIN = 0.1
L_REG = 1e-4

_mesh = plsc.VectorSubcoreMesh(core_axis_name="c", subcore_axis_name="s")


@functools.partial(
    pl.kernel,
    out_type=jax.ShapeDtypeStruct((NN, D), jnp.float32),
    mesh=_mesh,
    compiler_params=pltpu.CompilerParams(use_tc_tiling_on_sc=False),
    scratch_types=[
        pltpu.VMEM((G, CH), jnp.int32),
        pltpu.VMEM((G, CH), jnp.int32),
        pltpu.VMEM((G, CH), jnp.float32),
        pltpu.VMEM((4, CH, D), jnp.float32),
        pltpu.VMEM((WB, D), jnp.float32),
        pltpu.VMEM_SHARED((NU, D), jnp.float32),
        [pltpu.SemaphoreType.DMA] * 4,
        [pltpu.SemaphoreType.DMA] * 4,
    ],
)
def _spmm(row2d, col2d, w2d, prev, out, rowg, colg, wg, rows, wrb, acc,
          gsems, ssems):
    cid = lax.axis_index("c")
    sid = lax.axis_index("s")

    zero = jnp.zeros((16,), jnp.float32)

    def _z1(i, _):
        wrb[i // 2, pl.ds((i % 2) * 16, 16)] = zero
        return 0

    lax.fori_loop(0, WB * 2, _z1, 0)

    nwb = 24 + jnp.where(sid < NWB - 24 * NS, 1, 0)

    def _z2(j, _):
        pltpu.sync_copy(wrb, acc.at[pl.ds((sid + j * NS) * WB, WB), :])
        return 0

    lax.fori_loop(0, nwb, _z2, 0)

    @pl.when(sid == NS - 1)
    def _():
        pltpu.sync_copy(wrb.at[pl.ds(0, WB_TAIL), :],
                        acc.at[pl.ds(NWB * WB, WB_TAIL), :])

    plsc.subcore_barrier()

    n = 390 + jnp.where(sid < NCH - 390 * NS, 1, 0)
    base_c = sid * 390 + jnp.minimum(sid, NCH - 390 * NS)
    off = cid * NU

    def _scale(j, p):
        def _body(s, _):
            wv = wg[j, pl.ds(s * 16, 16)]
            for q in range(16):
                wq = wv[q]
                e = s * 16 + q
                rows[p, e, pl.ds(0, 16)] = rows[p, e, pl.ds(0, 16)] * wq
                rows[p, e, pl.ds(16, 16)] = rows[p, e, pl.ds(16, 16)] * wq
            return 0
        lax.fori_loop(0, CH // 16, _body, 0)

    def _localize(ng):
        def _body(i, _):
            r = i // (CH // 16)
            s = (i % (CH // 16)) * 16
            rowg[r, pl.ds(s, 16)] = rowg[r, pl.ds(s, 16)] - off
            return 0
        lax.fori_loop(0, ng * (CH // 16), _body, 0)

    def _group(gi, _):
        cg = cid * NCH + base_c + gi * G
        pltpu.sync_copy(row2d.at[pl.ds(cg, G), :], rowg)
        pltpu.sync_copy(col2d.at[pl.ds(cg, G), :], colg)
        pltpu.sync_copy(w2d.at[pl.ds(cg, G), :], wg)
        _localize(G)
        gh = {}
        sh = {}
        gh[0] = pltpu.async_copy(prev.at[colg.at[0]], rows.at[0], gsems[0])
        for j in range(G):
            if j + 1 < G:
                if j + 1 >= 4:
                    sh[j + 1 - 4].wait()
                m = (j + 1) % 4
                gh[j + 1] = pltpu.async_copy(prev.at[colg.at[j + 1]],
                                             rows.at[m], gsems[m])
            gh[j].wait()
            _scale(j, j % 4)
            sh[j] = pltpu.async_copy(rows.at[j % 4], acc.at[rowg.at[j]],
                                     ssems[j % 4], add=True)
        for j in range(G - 4, G):
            sh[j].wait()
        return 0

    lax.fori_loop(0, NFG, _group, 0)

    def _tail(k, _):
        cg = cid * NCH + base_c + NFG * G + k
        pltpu.sync_copy(row2d.at[cg], rowg.at[0])
        pltpu.sync_copy(col2d.at[cg], colg.at[0])
        pltpu.sync_copy(w2d.at[cg], wg.at[0])
        _localize(1)
        pltpu.async_copy(prev.at[colg.at[0]], rows.at[0], gsems[0]).wait()
        _scale(0, 0)
        pltpu.sync_copy(rows.at[0], acc.at[rowg.at[0]], add=True)
        return 0

    lax.fori_loop(0, n - NFG * G, _tail, 0)
    plsc.subcore_barrier()

    def _wbk(j, _):
        r0 = (sid + j * NS) * WB
        pltpu.sync_copy(acc.at[pl.ds(r0, WB), :], wrb)
        pltpu.sync_copy(wrb, out.at[pl.ds(cid * NU + r0, WB), :])
        return 0

    lax.fori_loop(0, nwb, _wbk, 0)

    @pl.when(sid == NS - 1)
    def _():
        pltpu.sync_copy(acc.at[pl.ds(NWB * WB, WB_TAIL), :],
                        wrb.at[pl.ds(0, WB_TAIL), :])
        pltpu.sync_copy(wrb.at[pl.ds(0, WB_TAIL), :],
                        out.at[pl.ds(cid * NU + NWB * WB, WB_TAIL), :])


_BPT = B // (2 * NS)


@functools.partial(
    pl.kernel,
    out_type=[jax.ShapeDtypeStruct((B, D), jnp.float32) for _ in range(5)],
    mesh=_mesh,
    compiler_params=pltpu.CompilerParams(use_tc_tiling_on_sc=False),
    scratch_types=[
        pltpu.VMEM((_BPT,), jnp.int32),
        pltpu.VMEM((_BPT, D), jnp.float32),
        pltpu.VMEM((_BPT, D), jnp.float32),
        pltpu.SemaphoreType.DMA,
    ],
)
def _gather_mean(e0, e1, e2, e3, user, positive, negative,
                 user_e, pos_e, ego_u, ego_p, ego_n, idxb, rb, accb, sem):
    cid = lax.axis_index("c")
    sid = lax.axis_index("s")
    wid = sid * 2 + cid
    base = wid * _BPT
    tables = [e0, e1, e2, e3]

    def _acc_from_rb(first):
        def _body(i, _):
            r = i // 2
            s = (i % 2) * 16
            v = rb[r, pl.ds(s, 16)]
            if first:
                accb[r, pl.ds(s, 16)] = v
            else:
                accb[r, pl.ds(s, 16)] = accb[r, pl.ds(s, 16)] + v
            return 0
        lax.fori_loop(0, _BPT * 2, _body, 0)

    def _scale_acc():
        def _body(i, _):
            r = i // 2
            s = (i % 2) * 16
            accb[r, pl.ds(s, 16)] = accb[r, pl.ds(s, 16)] * 0.25
            return 0
        lax.fori_loop(0, _BPT * 2, _body, 0)

    def _shift_idx(delta):
        def _body(i, _):
            idxb[pl.ds(i * 16, 16)] = idxb[pl.ds(i * 16, 16)] + delta
            return 0
        lax.fori_loop(0, _BPT // 16, _body, 0)

    pltpu.sync_copy(user.at[pl.ds(base, _BPT)], idxb)
    for li, t in enumerate(tables):
        pltpu.async_copy(t.at[idxb], rb, sem).wait()
        if li == 0:
            pltpu.sync_copy(rb, ego_u.at[pl.ds(base, _BPT), :])
        _acc_from_rb(first=(li == 0))
    _scale_acc()
    pltpu.sync_copy(accb, user_e.at[pl.ds(base, _BPT), :])

    pltpu.sync_copy(positive.at[pl.ds(base, _BPT)], idxb)
    _shift_idx(NU)
    for li, t in enumerate(tables):
        pltpu.async_copy(t.at[idxb], rb, sem).wait()
        if li == 0:
            pltpu.sync_copy(rb, ego_p.at[pl.ds(base, _BPT), :])
        _acc_from_rb(first=(li == 0))
    _scale_acc()
    pltpu.sync_copy(accb, pos_e.at[pl.ds(base, _BPT), :])

    pltpu.sync_copy(negative.at[pl.ds(base, _BPT)], idxb)
    _shift_idx(NU)
    pltpu.async_copy(e0.at[idxb], rb, sem).wait()
    pltpu.sync_copy(rb, ego_n.at[pl.ds(base, _BPT), :])


def _loss_body(ue_b, pe_b, ue_f, pe_f, eu, ep, en, reg_ref, na_ref):
    i = pl.program_id(0)

    def _nrm(x):
        n = jnp.maximum(jnp.sqrt(jnp.sum(x * x, axis=1, keepdims=True)), 1e-12)
        return x / n

    e1nb = _nrm(ue_b[...])
    e2nb = _nrm(pe_b[...])
    bfull = _nrm(ue_f[...]) + _nrm(pe_f[...])
    t = lax.dot_general(e1nb, bfull, (((1,), (1,)), ((), ())),
                        preferred_element_type=jnp.float32,
                        precision=lax.Precision.HIGHEST)
    f = jnp.exp(t * INV_T) + jnp.exp(jnp.maximum(t - MARGIN, 0.0) * INV_T)
    tot = jnp.sum(f, axis=1)
    sim = jnp.sum(e1nb * e2nb, axis=1)
    pos = jnp.exp(sim * INV_T) + jnp.exp(jnp.maximum(sim - MARGIN, 0.0) * INV_T)
    part = jnp.sum(-jnp.log(pos / tot + 1e-5))

    @pl.when(i == 0)
    def _():
        na_ref[...] = jnp.zeros((1, 1), jnp.float32)

    na_ref[...] = na_ref[...] + part.reshape(1, 1)

    @pl.when(i == NBLK - 1)
    def _():
        na_ref[...] = na_ref[...] * (1.0 / B)
        reg = (L_REG * 0.5 / B) * (
            jnp.sum(eu[...] ** 2) + jnp.sum(ep[...] ** 2) + jnp.sum(en[...] ** 2))
        reg_ref[...] = reg.reshape(1, 1)


def _loss_tc(ue, pe, eu, ep, en):
    full = pl.BlockSpec((B, D), lambda i: (0, 0))
    blk = pl.BlockSpec((BR, D), lambda i: (i, 0))
    scal = pl.BlockSpec((1, 1), lambda i: (0, 0))
    return pl.pallas_call(
        _loss_body,
        grid=(NBLK,),
        in_specs=[blk, blk, full, full, full, full, full],
        out_specs=[scal, scal],
        out_shape=[jax.ShapeDtypeStruct((1, 1), jnp.float32),
                   jax.ShapeDtypeStruct((1, 1), jnp.float32)],
    )(ue, pe, ue, pe, eu, ep, en)


def kernel(user, positive, negative, edge_index, edge_weight, user_emb_w, item_emb_w):
    e0 = jnp.concatenate([user_emb_w, item_emb_w], axis=0)
    row2d = edge_index[0].reshape(2 * NCH, CH)
    col2d = edge_index[1].reshape(2 * NCH, CH)
    w2d = edge_weight.reshape(2 * NCH, CH)
    e1 = _spmm(row2d, col2d, w2d, e0)
    e2 = _spmm(row2d, col2d, w2d, e1)
    e3 = _spmm(row2d, col2d, w2d, e2)
    ue, pe, eu, ep, en = _gather_mean(e0, e1, e2, e3, user, positive, negative)
    reg, na = _loss_tc(ue, pe, eu, ep, en)
    return (reg[0, 0], na[0, 0])
